# baseline (speedup 1.0000x reference)
import functools

import jax
import jax.numpy as jnp
from jax import lax
from jax.experimental import pallas as pl
from jax.experimental.pallas import tpu as pltpu
from jax.experimental.pallas import tpu_sc as plsc

N = 10000
K = 8
NUM_CLASSES = 40

NB = 128
NBLK = 79
NP = NB * NBLK
NW = 32
CHUNK = 128
CHUNKS_PW = 20
NE_PAD = NW * CHUNKS_PW * CHUNK

_HI = jax.lax.Precision.HIGHEST


def _knn_body(xp_ref, xpt_ref, wu_ref, wv_ref, nbr_ref, u_ref, v_ref, d_ref):
    b = pl.program_id(0)
    q = xp_ref[...]
    xt = xpt_ref[...]
    sq = jnp.sum(xt * xt, axis=0, keepdims=True)
    sqq = jnp.sum(q * q, axis=1, keepdims=True)
    d = sqq + sq - 2.0 * jnp.dot(q, xt, preferred_element_type=jnp.float32,
                                 precision=_HI)
    inf = jnp.float32(jnp.inf)
    d_ref[...] = d
    lane = lax.broadcasted_iota(jnp.int32, (NB, 128), 1)
    rloc = lax.broadcasted_iota(jnp.int32, (NB, 128), 0)
    dcol = pl.multiple_of(b * 128, 128)
    dblk = d_ref[:, pl.ds(dcol, 128)]
    d_ref[:, pl.ds(dcol, 128)] = jnp.where(lane == rloc, inf, dblk)

    NT = NP // 128
    lanef = lane.astype(jnp.float32)
    nbr = jnp.zeros((NB, 128), jnp.float32)
    idx_i = None
    for k in range(K):
        V = jnp.full((NB, 128), inf, jnp.float32)
        T = jnp.zeros((NB, 128), jnp.float32)
        for t in range(NT):
            x = d_ref[:, t * 128:(t + 1) * 128]
            if k > 0:
                x = jnp.where((lane + t * 128) == idx_i, inf, x)
                if k < K - 1:
                    d_ref[:, t * 128:(t + 1) * 128] = x
            mlt = x < V
            T = jnp.where(mlt, jnp.float32(t), T)
            V = jnp.where(mlt, x, V)
        mn = jnp.min(V, axis=1, keepdims=True)
        gidx = T * 128.0 + lanef
        cand = jnp.where(V == mn, gidx, jnp.float32(1e9))
        idxf = jnp.min(cand, axis=1, keepdims=True)
        nbr = jnp.where(lane == k, idxf, nbr)
        idx_i = idxf.astype(jnp.int32)
    nbr_ref[...] = nbr.astype(jnp.int32)
    u_ref[...] = jnp.dot(q, wu_ref[...], preferred_element_type=jnp.float32,
                         precision=_HI)
    v_ref[...] = jnp.dot(q, wv_ref[...], preferred_element_type=jnp.float32,
                         precision=_HI)


def _l1_body(e1_ref, u_ref, v_ref, xp_ref, w2_ref, w3a_ref, w3b_ref,
             b1_ref, b2_ref, g_ref, w_ref):
    v = v_ref[...]
    b1 = b1_ref[...]
    w2 = w2_ref[...]
    acc = jnp.dot(jnp.maximum(u_ref[:, :32] - v + b1, 0.0), w2,
                  preferred_element_type=jnp.float32, precision=_HI)
    for k in range(K):
        uk = e1_ref[:, k * 128:k * 128 + 32]
        hk = jnp.dot(jnp.maximum(uk - v + b1, 0.0), w2,
                     preferred_element_type=jnp.float32, precision=_HI)
        acc = jnp.maximum(acc, hk)
    h1 = jnp.maximum(acc + b2_ref[...], 0.0)
    wv = jnp.dot(xp_ref[...], w3b_ref[...], preferred_element_type=jnp.float32,
                 precision=_HI)
    g_ref[...] = jnp.dot(h1, w3a_ref[...], preferred_element_type=jnp.float32,
                         precision=_HI) + wv
    w_ref[...] = wv[:, :64]


def _l2_body(e2_ref, g_ref, w_ref, w4_ref, wc_ref, b3_ref, b4_ref, bc_ref,
             out_ref):
    w = w_ref[...]
    b3 = b3_ref[...]
    w4 = w4_ref[...]
    acc = jnp.dot(jnp.maximum(g_ref[:, :64] - w + b3, 0.0), w4,
                  preferred_element_type=jnp.float32, precision=_HI)
    for k in range(K):
        gk = e2_ref[:, k * 128:k * 128 + 64]
        hk = jnp.dot(jnp.maximum(gk - w + b3, 0.0), w4,
                     preferred_element_type=jnp.float32, precision=_HI)
        acc = jnp.maximum(acc, hk)
    h2 = jnp.maximum(acc + b4_ref[...], 0.0)
    logits = jnp.dot(h2, wc_ref[...], preferred_element_type=jnp.float32,
                     precision=_HI) + bc_ref[...]
    colk = lax.broadcasted_iota(jnp.int32, (NB, 128), 1)
    valid = colk < NUM_CLASSES
    lm = jnp.where(valid, logits, jnp.float32(-1e30))
    m = jnp.max(lm, axis=1, keepdims=True)
    e = jnp.where(valid, jnp.exp(logits - m), 0.0)
    s = jnp.sum(e, axis=1, keepdims=True)
    out_ref[...] = logits - m - jnp.log(s)


def _sc_gather(table, idx3d, d):
    mesh = plsc.VectorSubcoreMesh(core_axis_name="c", subcore_axis_name="s")
    NBUF = 4

    @functools.partial(
        pl.kernel, mesh=mesh,
        out_type=jax.ShapeDtypeStruct((NE_PAD, d), jnp.float32),
        scratch_types=[
            pltpu.VMEM((CHUNKS_PW, CHUNK), jnp.int32),
            pltpu.VMEM((NBUF, CHUNK, d), jnp.float32),
            pltpu.SemaphoreType.DMA((NBUF,)),
            pltpu.SemaphoreType.DMA((NBUF,)),
        ],
    )
    def k(table_hbm, idx_hbm, out_hbm, idx_v, rows, gsem, wsem):
        wid = lax.axis_index("s") * 2 + lax.axis_index("c")
        base = wid * CHUNKS_PW
        pltpu.sync_copy(idx_hbm.at[wid], idx_v)
        gh = [None] * CHUNKS_PW
        wh = [None] * CHUNKS_PW
        for j in range(CHUNKS_PW):
            b = j % NBUF
            if j >= NBUF:
                wh[j - NBUF].wait()
            gh[j] = pltpu.async_copy(table_hbm.at[idx_v.at[j]], rows.at[b],
                                     gsem.at[b])
            i = j - (NBUF - 1)
            if i >= 0:
                gh[i].wait()
                wh[i] = pltpu.async_copy(
                    rows.at[i % NBUF],
                    out_hbm.at[pl.ds((base + i) * CHUNK, CHUNK)],
                    wsem.at[i % NBUF])
        for i in range(CHUNKS_PW - (NBUF - 1), CHUNKS_PW):
            gh[i].wait()
            wh[i] = pltpu.async_copy(
                rows.at[i % NBUF],
                out_hbm.at[pl.ds((base + i) * CHUNK, CHUNK)],
                wsem.at[i % NBUF])
        for i in range(CHUNKS_PW - NBUF, CHUNKS_PW):
            wh[i].wait()

    return k(table, idx3d)


def _blk(shape, imap):
    return pl.BlockSpec(shape, imap)


def kernel(x, batch, W1, b1, W2, b2, W3, b3, W4, b4, Wc, bc):
    f32 = jnp.float32
    xp = jnp.zeros((NP, 8), f32).at[:N, :3].set(x)
    xpt = jnp.concatenate(
        [jnp.concatenate([x.T, jnp.full((3, NP - N), 1e18, f32)], axis=1),
         jnp.zeros((5, NP), f32)], axis=0)
    Wu = jnp.zeros((8, 128), f32).at[:3, :32].set(W1[:3] + W1[3:6])
    Wv = jnp.zeros((8, 32), f32).at[:3].set(W1[3:6])
    W3a = jnp.zeros((64, 128), f32).at[:, :64].set(W3[:64])
    W3bp = jnp.zeros((8, 128), f32).at[:3, :64].set(W3[64:67])
    Wcp = jnp.zeros((128, 128), f32).at[:, :NUM_CLASSES].set(Wc)
    bcp = jnp.zeros((128,), f32).at[:NUM_CLASSES].set(bc)

    nbr_f, u, v = pl.pallas_call(
        _knn_body,
        grid=(NBLK,),
        in_specs=[
            _blk((NB, 8), lambda b: (b, 0)),
            _blk((8, NP), lambda b: (0, 0)),
            _blk((8, 128), lambda b: (0, 0)),
            _blk((8, 32), lambda b: (0, 0)),
        ],
        out_specs=[
            _blk((NB, 128), lambda b: (b, 0)),
            _blk((NB, 128), lambda b: (b, 0)),
            _blk((NB, 32), lambda b: (b, 0)),
        ],
        out_shape=[
            jax.ShapeDtypeStruct((NP, 128), jnp.int32),
            jax.ShapeDtypeStruct((NP, 128), f32),
            jax.ShapeDtypeStruct((NP, 32), f32),
        ],
        scratch_shapes=[pltpu.VMEM((NB, NP), jnp.float32)],
    )(xp, xpt, Wu, Wv)

    idx = nbr_f[:N, :K].reshape(-1)
    idx = jnp.concatenate(
        [idx, jnp.zeros((NE_PAD - N * K,), jnp.int32)]).reshape(
            NW, CHUNKS_PW, CHUNK)

    e1 = _sc_gather(u, idx, 128).reshape(-1, K * 128)
    g, w = pl.pallas_call(
        _l1_body,
        grid=(NBLK,),
        in_specs=[
            _blk((NB, K * 128), lambda b: (b, 0)),
            _blk((NB, 128), lambda b: (b, 0)),
            _blk((NB, 32), lambda b: (b, 0)),
            _blk((NB, 8), lambda b: (b, 0)),
            _blk((32, 64), lambda b: (0, 0)),
            _blk((64, 128), lambda b: (0, 0)),
            _blk((8, 128), lambda b: (0, 0)),
            _blk((1, 32), lambda b: (0, 0)),
            _blk((1, 64), lambda b: (0, 0)),
        ],
        out_specs=[
            _blk((NB, 128), lambda b: (b, 0)),
            _blk((NB, 64), lambda b: (b, 0)),
        ],
        out_shape=[
            jax.ShapeDtypeStruct((NP, 128), f32),
            jax.ShapeDtypeStruct((NP, 64), f32),
        ],
    )(e1, u, v, xp, W2, W3a, W3bp, b1.reshape(1, 32), b2.reshape(1, 64))

    e2 = _sc_gather(g, idx, 128).reshape(-1, K * 128)
    out = pl.pallas_call(
        _l2_body,
        grid=(NBLK,),
        in_specs=[
            _blk((NB, K * 128), lambda b: (b, 0)),
            _blk((NB, 128), lambda b: (b, 0)),
            _blk((NB, 64), lambda b: (b, 0)),
            _blk((64, 128), lambda b: (0, 0)),
            _blk((128, 128), lambda b: (0, 0)),
            _blk((1, 64), lambda b: (0, 0)),
            _blk((1, 128), lambda b: (0, 0)),
            _blk((1, 128), lambda b: (0, 0)),
        ],
        out_specs=_blk((NB, 128), lambda b: (b, 0)),
        out_shape=jax.ShapeDtypeStruct((NP, 128), f32),
    )(e2, g, w, W4, Wcp, b3.reshape(1, 64), b4.reshape(1, 128),
      bcp.reshape(1, 128))

    return out[:N, :NUM_CLASSES]


# speedup vs baseline: 1.0513x; 1.0513x over previous
import functools

import jax
import jax.numpy as jnp
from jax import lax
from jax.experimental import pallas as pl
from jax.experimental.pallas import tpu as pltpu
from jax.experimental.pallas import tpu_sc as plsc

N = 10000
K = 8
NUM_CLASSES = 40

NB = 128
NBLK = 79
NP = NB * NBLK
NW = 32
CHUNK = 128
CHUNKS_PW = 20
NE_PAD = NW * CHUNKS_PW * CHUNK

_HI = jax.lax.Precision.HIGHEST


def _knn_body(xp_ref, xpt_ref, wu_ref, wv_ref, nbr_ref, u_ref, v_ref, d_ref):
    b = pl.program_id(0)
    q = xp_ref[...]
    xt = xpt_ref[...]
    sq = jnp.sum(xt * xt, axis=0, keepdims=True)
    sqq = jnp.sum(q * q, axis=1, keepdims=True)
    d = sqq + sq - 2.0 * jnp.dot(q, xt, preferred_element_type=jnp.float32,
                                 precision=_HI)
    inf = jnp.float32(jnp.inf)
    coli = lax.broadcasted_iota(jnp.int32, (NB, NP), 1)
    rowg = lax.broadcasted_iota(jnp.int32, (NB, NP), 0) + (b * NB)
    d_ref[...] = jnp.where((coli == rowg) | (coli >= N), inf, d)
    lane = lax.broadcasted_iota(jnp.int32, (NB, 128), 1)

    NT = NP // 128
    lanef = lane.astype(jnp.float32)
    nbr = jnp.zeros((NB, 128), jnp.float32)
    idx_i = None
    for k in range(K):
        V = jnp.full((NB, 128), inf, jnp.float32)
        T = jnp.zeros((NB, 128), jnp.float32)
        for t in range(NT):
            x = d_ref[:, t * 128:(t + 1) * 128]
            if k > 0:
                x = jnp.where((lane + t * 128) == idx_i, inf, x)
                if k < K - 1:
                    d_ref[:, t * 128:(t + 1) * 128] = x
            mlt = x < V
            T = jnp.where(mlt, jnp.float32(t), T)
            V = jnp.where(mlt, x, V)
        mn = jnp.min(V, axis=1, keepdims=True)
        gidx = T * 128.0 + lanef
        cand = jnp.where(V == mn, gidx, jnp.float32(1e9))
        idxf = jnp.min(cand, axis=1, keepdims=True)
        nbr = jnp.where(lane == k, idxf, nbr)
        idx_i = idxf.astype(jnp.int32)
    nbr_ref[...] = nbr.astype(jnp.int32)
    u_ref[...] = jnp.dot(q, wu_ref[...], preferred_element_type=jnp.float32,
                         precision=_HI)
    v_ref[...] = jnp.dot(q, wv_ref[...], preferred_element_type=jnp.float32,
                         precision=_HI)


def _l1_body(e1_ref, u_ref, v_ref, xp_ref, w2_ref, w3a_ref, w3b_ref,
             b1_ref, b2_ref, g_ref, w_ref):
    v = v_ref[...]
    b1 = b1_ref[...]
    w2 = w2_ref[...]
    acc = jnp.dot(jnp.maximum(u_ref[:, :32] - v + b1, 0.0), w2,
                  preferred_element_type=jnp.float32, precision=_HI)
    for k in range(K):
        uk = e1_ref[:, k * 128:k * 128 + 32]
        hk = jnp.dot(jnp.maximum(uk - v + b1, 0.0), w2,
                     preferred_element_type=jnp.float32, precision=_HI)
        acc = jnp.maximum(acc, hk)
    h1 = jnp.maximum(acc + b2_ref[...], 0.0)
    wv = jnp.dot(xp_ref[...], w3b_ref[...], preferred_element_type=jnp.float32,
                 precision=_HI)
    g_ref[...] = jnp.dot(h1, w3a_ref[...], preferred_element_type=jnp.float32,
                         precision=_HI) + wv
    w_ref[...] = wv[:, :64]


def _l2_body(e2_ref, g_ref, w_ref, w4_ref, wc_ref, b3_ref, b4_ref, bc_ref,
             out_ref):
    w = w_ref[...]
    b3 = b3_ref[...]
    w4 = w4_ref[...]
    acc = jnp.dot(jnp.maximum(g_ref[:, :64] - w + b3, 0.0), w4,
                  preferred_element_type=jnp.float32, precision=_HI)
    for k in range(K):
        gk = e2_ref[:, k * 128:k * 128 + 64]
        hk = jnp.dot(jnp.maximum(gk - w + b3, 0.0), w4,
                     preferred_element_type=jnp.float32, precision=_HI)
        acc = jnp.maximum(acc, hk)
    h2 = jnp.maximum(acc + b4_ref[...], 0.0)
    logits = jnp.dot(h2, wc_ref[...], preferred_element_type=jnp.float32,
                     precision=_HI) + bc_ref[...]
    colk = lax.broadcasted_iota(jnp.int32, (NB, 128), 1)
    valid = colk < NUM_CLASSES
    lm = jnp.where(valid, logits, jnp.float32(-1e30))
    m = jnp.max(lm, axis=1, keepdims=True)
    e = jnp.where(valid, jnp.exp(logits - m), 0.0)
    s = jnp.sum(e, axis=1, keepdims=True)
    out_ref[...] = logits - m - jnp.log(s)


def _sc_gather(table, idx3d, d):
    mesh = plsc.VectorSubcoreMesh(core_axis_name="c", subcore_axis_name="s")
    NBUF = 4

    @functools.partial(
        pl.kernel, mesh=mesh,
        out_type=jax.ShapeDtypeStruct((NE_PAD, d), jnp.float32),
        scratch_types=[
            pltpu.VMEM((CHUNKS_PW, CHUNK), jnp.int32),
            pltpu.VMEM((NBUF, CHUNK, d), jnp.float32),
            pltpu.SemaphoreType.DMA((NBUF,)),
            pltpu.SemaphoreType.DMA((NBUF,)),
        ],
    )
    def k(table_hbm, idx_hbm, out_hbm, idx_v, rows, gsem, wsem):
        wid = lax.axis_index("s") * 2 + lax.axis_index("c")
        base = wid * CHUNKS_PW
        pltpu.sync_copy(idx_hbm.at[wid], idx_v)
        gh = [None] * CHUNKS_PW
        wh = [None] * CHUNKS_PW
        for j in range(CHUNKS_PW):
            b = j % NBUF
            if j >= NBUF:
                wh[j - NBUF].wait()
            gh[j] = pltpu.async_copy(table_hbm.at[idx_v.at[j]], rows.at[b],
                                     gsem.at[b])
            i = j - (NBUF - 1)
            if i >= 0:
                gh[i].wait()
                wh[i] = pltpu.async_copy(
                    rows.at[i % NBUF],
                    out_hbm.at[pl.ds((base + i) * CHUNK, CHUNK)],
                    wsem.at[i % NBUF])
        for i in range(CHUNKS_PW - (NBUF - 1), CHUNKS_PW):
            gh[i].wait()
            wh[i] = pltpu.async_copy(
                rows.at[i % NBUF],
                out_hbm.at[pl.ds((base + i) * CHUNK, CHUNK)],
                wsem.at[i % NBUF])
        for i in range(CHUNKS_PW - NBUF, CHUNKS_PW):
            wh[i].wait()

    return k(table, idx3d)


def _blk(shape, imap):
    return pl.BlockSpec(shape, imap)


def kernel(x, batch, W1, b1, W2, b2, W3, b3, W4, b4, Wc, bc):
    f32 = jnp.float32
    xp = jnp.zeros((NP, 8), f32).at[:N, :3].set(x)
    xpt = jnp.concatenate(
        [jnp.concatenate([x.T, jnp.full((3, NP - N), 1e18, f32)], axis=1),
         jnp.zeros((5, NP), f32)], axis=0)
    Wu = jnp.zeros((8, 128), f32).at[:3, :32].set(W1[:3] + W1[3:6])
    Wv = jnp.zeros((8, 32), f32).at[:3].set(W1[3:6])
    W3a = jnp.zeros((64, 128), f32).at[:, :64].set(W3[:64])
    W3bp = jnp.zeros((8, 128), f32).at[:3, :64].set(W3[64:67])
    Wcp = jnp.zeros((128, 128), f32).at[:, :NUM_CLASSES].set(Wc)
    bcp = jnp.zeros((128,), f32).at[:NUM_CLASSES].set(bc)

    nbr_f, u, v = pl.pallas_call(
        _knn_body,
        grid=(NBLK,),
        in_specs=[
            _blk((NB, 8), lambda b: (b, 0)),
            _blk((8, NP), lambda b: (0, 0)),
            _blk((8, 128), lambda b: (0, 0)),
            _blk((8, 32), lambda b: (0, 0)),
        ],
        out_specs=[
            _blk((NB, 128), lambda b: (b, 0)),
            _blk((NB, 128), lambda b: (b, 0)),
            _blk((NB, 32), lambda b: (b, 0)),
        ],
        out_shape=[
            jax.ShapeDtypeStruct((NP, 128), jnp.int32),
            jax.ShapeDtypeStruct((NP, 128), f32),
            jax.ShapeDtypeStruct((NP, 32), f32),
        ],
        scratch_shapes=[pltpu.VMEM((NB, NP), jnp.float32)],
    )(xp, xpt, Wu, Wv)

    idx = nbr_f[:N, :K].reshape(-1)
    idx = jnp.concatenate(
        [idx, jnp.zeros((NE_PAD - N * K,), jnp.int32)]).reshape(
            NW, CHUNKS_PW, CHUNK)

    e1 = _sc_gather(u, idx, 128).reshape(-1, K * 128)
    g, w = pl.pallas_call(
        _l1_body,
        grid=(NBLK,),
        in_specs=[
            _blk((NB, K * 128), lambda b: (b, 0)),
            _blk((NB, 128), lambda b: (b, 0)),
            _blk((NB, 32), lambda b: (b, 0)),
            _blk((NB, 8), lambda b: (b, 0)),
            _blk((32, 64), lambda b: (0, 0)),
            _blk((64, 128), lambda b: (0, 0)),
            _blk((8, 128), lambda b: (0, 0)),
            _blk((1, 32), lambda b: (0, 0)),
            _blk((1, 64), lambda b: (0, 0)),
        ],
        out_specs=[
            _blk((NB, 128), lambda b: (b, 0)),
            _blk((NB, 64), lambda b: (b, 0)),
        ],
        out_shape=[
            jax.ShapeDtypeStruct((NP, 128), f32),
            jax.ShapeDtypeStruct((NP, 64), f32),
        ],
    )(e1, u, v, xp, W2, W3a, W3bp, b1.reshape(1, 32), b2.reshape(1, 64))

    e2 = _sc_gather(g, idx, 128).reshape(-1, K * 128)
    out = pl.pallas_call(
        _l2_body,
        grid=(NBLK,),
        in_specs=[
            _blk((NB, K * 128), lambda b: (b, 0)),
            _blk((NB, 128), lambda b: (b, 0)),
            _blk((NB, 64), lambda b: (b, 0)),
            _blk((64, 128), lambda b: (0, 0)),
            _blk((128, 128), lambda b: (0, 0)),
            _blk((1, 64), lambda b: (0, 0)),
            _blk((1, 128), lambda b: (0, 0)),
            _blk((1, 128), lambda b: (0, 0)),
        ],
        out_specs=_blk((NB, 128), lambda b: (b, 0)),
        out_shape=jax.ShapeDtypeStruct((NP, 128), f32),
    )(e2, g, w, W4, Wcp, b3.reshape(1, 64), b4.reshape(1, 128),
      bcp.reshape(1, 128))

    return out[:N, :NUM_CLASSES]


# speedup vs baseline: 1.2097x; 1.1507x over previous
import functools

import jax
import jax.numpy as jnp
from jax import lax
from jax.experimental import pallas as pl
from jax.experimental.pallas import tpu as pltpu
from jax.experimental.pallas import tpu_sc as plsc

N = 10000
K = 8
NUM_CLASSES = 40

NB = 128
NBLK = 79
NP = NB * NBLK
NW = 32
CHUNK = 128
CHUNKS_PW = 20
NE_PAD = NW * CHUNKS_PW * CHUNK

_HI = jax.lax.Precision.HIGHEST


def _knn_body(xp_ref, xpt_ref, wu_ref, wv_ref, nbr_ref, u_ref, v_ref, d_ref):
    b = pl.program_id(0)
    q = xp_ref[...]
    xt = xpt_ref[...]
    sq = jnp.sum(xt * xt, axis=0, keepdims=True)
    sqq = jnp.sum(q * q, axis=1, keepdims=True)
    d = sqq + sq - 2.0 * jnp.dot(q, xt, preferred_element_type=jnp.float32,
                                 precision=_HI)
    inf = jnp.float32(jnp.inf)
    coli = lax.broadcasted_iota(jnp.int32, (NB, NP), 1)
    rowg = lax.broadcasted_iota(jnp.int32, (NB, NP), 0) + (b * NB)
    d_ref[...] = jnp.where((coli == rowg) | (coli >= N), inf, d)
    lane = lax.broadcasted_iota(jnp.int32, (NB, 128), 1)

    NT = NP // 128
    lanef = lane.astype(jnp.float32)
    nbr = jnp.zeros((NB, 128), jnp.float32)
    idx_i = None
    for k in range(K):
        V = jnp.full((NB, 128), inf, jnp.float32)
        T = jnp.zeros((NB, 128), jnp.float32)
        for t in range(NT):
            x = d_ref[:, t * 128:(t + 1) * 128]
            if k > 0:
                x = jnp.where((lane + t * 128) == idx_i, inf, x)
                if k < K - 1:
                    d_ref[:, t * 128:(t + 1) * 128] = x
            mlt = x < V
            T = jnp.where(mlt, jnp.float32(t), T)
            V = jnp.where(mlt, x, V)
        mn = jnp.min(V, axis=1, keepdims=True)
        gidx = T * 128.0 + lanef
        cand = jnp.where(V == mn, gidx, jnp.float32(1e9))
        idxf = jnp.min(cand, axis=1, keepdims=True)
        nbr = jnp.where(lane == k, idxf, nbr)
        idx_i = idxf.astype(jnp.int32)
    nbr_ref[...] = nbr.astype(jnp.int32)
    u_ref[...] = jnp.dot(q, wu_ref[...], preferred_element_type=jnp.float32,
                         precision=_HI)
    v_ref[...] = jnp.dot(q, wv_ref[...], preferred_element_type=jnp.float32,
                         precision=_HI)


def _l1_body(e1_ref, u_ref, v_ref, xp_ref, w2_ref, w3a_ref, w3b_ref,
             b1_ref, b2_ref, g_ref, w_ref):
    v = v_ref[...]
    b1 = b1_ref[...]
    w2 = w2_ref[...]
    acc = jnp.dot(jnp.maximum(u_ref[...] - v + b1, 0.0), w2,
                  preferred_element_type=jnp.float32, precision=_HI)
    for k in range(K):
        uk = e1_ref[:, k * 32:(k + 1) * 32]
        hk = jnp.dot(jnp.maximum(uk - v + b1, 0.0), w2,
                     preferred_element_type=jnp.float32, precision=_HI)
        acc = jnp.maximum(acc, hk)
    h1 = jnp.maximum(acc + b2_ref[...], 0.0)
    wv = jnp.dot(xp_ref[...], w3b_ref[...], preferred_element_type=jnp.float32,
                 precision=_HI)
    g_ref[...] = jnp.dot(h1, w3a_ref[...], preferred_element_type=jnp.float32,
                         precision=_HI) + wv
    w_ref[...] = wv


def _l2_body(e2_ref, g_ref, w_ref, w4_ref, wc_ref, b3_ref, b4_ref, bc_ref,
             out_ref):
    w = w_ref[...]
    b3 = b3_ref[...]
    w4 = w4_ref[...]
    acc = jnp.dot(jnp.maximum(g_ref[...] - w + b3, 0.0), w4,
                  preferred_element_type=jnp.float32, precision=_HI)
    for k in range(K):
        gk = e2_ref[:, k * 64:(k + 1) * 64]
        hk = jnp.dot(jnp.maximum(gk - w + b3, 0.0), w4,
                     preferred_element_type=jnp.float32, precision=_HI)
        acc = jnp.maximum(acc, hk)
    h2 = jnp.maximum(acc + b4_ref[...], 0.0)
    logits = jnp.dot(h2, wc_ref[...], preferred_element_type=jnp.float32,
                     precision=_HI) + bc_ref[...]
    colk = lax.broadcasted_iota(jnp.int32, (NB, 128), 1)
    valid = colk < NUM_CLASSES
    lm = jnp.where(valid, logits, jnp.float32(-1e30))
    m = jnp.max(lm, axis=1, keepdims=True)
    e = jnp.where(valid, jnp.exp(logits - m), 0.0)
    s = jnp.sum(e, axis=1, keepdims=True)
    out_ref[...] = logits - m - jnp.log(s)


def _sc_gather(table, idx3d, d):
    mesh = plsc.VectorSubcoreMesh(core_axis_name="c", subcore_axis_name="s")
    NBUF = 4

    @functools.partial(
        pl.kernel, mesh=mesh,
        compiler_params=pltpu.CompilerParams(use_tc_tiling_on_sc=False),
        out_type=jax.ShapeDtypeStruct((NE_PAD, d), jnp.float32),
        scratch_types=[
            pltpu.VMEM((CHUNKS_PW, CHUNK), jnp.int32),
            pltpu.VMEM((NBUF, CHUNK, d), jnp.float32),
            pltpu.SemaphoreType.DMA((NBUF,)),
            pltpu.SemaphoreType.DMA((NBUF,)),
        ],
    )
    def k(table_hbm, idx_hbm, out_hbm, idx_v, rows, gsem, wsem):
        wid = lax.axis_index("s") * 2 + lax.axis_index("c")
        base = wid * CHUNKS_PW
        pltpu.sync_copy(idx_hbm.at[wid], idx_v)
        gh = [None] * CHUNKS_PW
        wh = [None] * CHUNKS_PW
        for j in range(CHUNKS_PW):
            b = j % NBUF
            if j >= NBUF:
                wh[j - NBUF].wait()
            gh[j] = pltpu.async_copy(table_hbm.at[idx_v.at[j]], rows.at[b],
                                     gsem.at[b])
            i = j - (NBUF - 1)
            if i >= 0:
                gh[i].wait()
                wh[i] = pltpu.async_copy(
                    rows.at[i % NBUF],
                    out_hbm.at[pl.ds((base + i) * CHUNK, CHUNK)],
                    wsem.at[i % NBUF])
        for i in range(CHUNKS_PW - (NBUF - 1), CHUNKS_PW):
            gh[i].wait()
            wh[i] = pltpu.async_copy(
                rows.at[i % NBUF],
                out_hbm.at[pl.ds((base + i) * CHUNK, CHUNK)],
                wsem.at[i % NBUF])
        for i in range(CHUNKS_PW - NBUF, CHUNKS_PW):
            wh[i].wait()

    return k(table, idx3d)


def _blk(shape, imap):
    return pl.BlockSpec(shape, imap)


def kernel(x, batch, W1, b1, W2, b2, W3, b3, W4, b4, Wc, bc):
    f32 = jnp.float32
    xp = jnp.zeros((NP, 8), f32).at[:N, :3].set(x)
    xpt = jnp.concatenate(
        [jnp.concatenate([x.T, jnp.full((3, NP - N), 1e18, f32)], axis=1),
         jnp.zeros((5, NP), f32)], axis=0)
    Wu = jnp.zeros((8, 32), f32).at[:3].set(W1[:3] + W1[3:6])
    Wv = jnp.zeros((8, 32), f32).at[:3].set(W1[3:6])
    W3a = W3[:64]
    W3bp = jnp.zeros((8, 64), f32).at[:3].set(W3[64:67])
    Wcp = jnp.zeros((128, 128), f32).at[:, :NUM_CLASSES].set(Wc)
    bcp = jnp.zeros((128,), f32).at[:NUM_CLASSES].set(bc)

    nbr_f, u, v = pl.pallas_call(
        _knn_body,
        grid=(NBLK,),
        in_specs=[
            _blk((NB, 8), lambda b: (b, 0)),
            _blk((8, NP), lambda b: (0, 0)),
            _blk((8, 32), lambda b: (0, 0)),
            _blk((8, 32), lambda b: (0, 0)),
        ],
        out_specs=[
            _blk((NB, 128), lambda b: (b, 0)),
            _blk((NB, 32), lambda b: (b, 0)),
            _blk((NB, 32), lambda b: (b, 0)),
        ],
        out_shape=[
            jax.ShapeDtypeStruct((NP, 128), jnp.int32),
            jax.ShapeDtypeStruct((NP, 32), f32),
            jax.ShapeDtypeStruct((NP, 32), f32),
        ],
        scratch_shapes=[pltpu.VMEM((NB, NP), jnp.float32)],
    )(xp, xpt, Wu, Wv)

    idx = nbr_f[:N, :K].reshape(-1)
    idx = jnp.concatenate(
        [idx, jnp.zeros((NE_PAD - N * K,), jnp.int32)]).reshape(
            NW, CHUNKS_PW, CHUNK)

    e1 = _sc_gather(u, idx, 32).reshape(-1, K * 32)
    g, w = pl.pallas_call(
        _l1_body,
        grid=(NBLK,),
        in_specs=[
            _blk((NB, K * 32), lambda b: (b, 0)),
            _blk((NB, 32), lambda b: (b, 0)),
            _blk((NB, 32), lambda b: (b, 0)),
            _blk((NB, 8), lambda b: (b, 0)),
            _blk((32, 64), lambda b: (0, 0)),
            _blk((64, 64), lambda b: (0, 0)),
            _blk((8, 64), lambda b: (0, 0)),
            _blk((1, 32), lambda b: (0, 0)),
            _blk((1, 64), lambda b: (0, 0)),
        ],
        out_specs=[
            _blk((NB, 64), lambda b: (b, 0)),
            _blk((NB, 64), lambda b: (b, 0)),
        ],
        out_shape=[
            jax.ShapeDtypeStruct((NP, 64), f32),
            jax.ShapeDtypeStruct((NP, 64), f32),
        ],
    )(e1, u, v, xp, W2, W3a, W3bp, b1.reshape(1, 32), b2.reshape(1, 64))

    e2 = _sc_gather(g, idx, 64).reshape(-1, K * 64)
    out = pl.pallas_call(
        _l2_body,
        grid=(NBLK,),
        in_specs=[
            _blk((NB, K * 64), lambda b: (b, 0)),
            _blk((NB, 64), lambda b: (b, 0)),
            _blk((NB, 64), lambda b: (b, 0)),
            _blk((64, 128), lambda b: (0, 0)),
            _blk((128, 128), lambda b: (0, 0)),
            _blk((1, 64), lambda b: (0, 0)),
            _blk((1, 128), lambda b: (0, 0)),
            _blk((1, 128), lambda b: (0, 0)),
        ],
        out_specs=_blk((NB, 128), lambda b: (b, 0)),
        out_shape=jax.ShapeDtypeStruct((NP, 128), f32),
    )(e2, g, w, W4, Wcp, b3.reshape(1, 64), b4.reshape(1, 128),
      bcp.reshape(1, 128))

    return out[:N, :NUM_CLASSES]


# speedup vs baseline: 1.2112x; 1.0012x over previous
import functools

import jax
import jax.numpy as jnp
from jax import lax
from jax.experimental import pallas as pl
from jax.experimental.pallas import tpu as pltpu
from jax.experimental.pallas import tpu_sc as plsc

N = 10000
K = 8
NUM_CLASSES = 40

NB = 128
NBLK = 79
NP = NB * NBLK
NW = 32
CHUNK = 128
CHUNKS_PW = 20
NE_PAD = NW * CHUNKS_PW * CHUNK

_HI = jax.lax.Precision.HIGHEST


def _knn_body(xp_ref, xpt_ref, wu_ref, wv_ref, nbr_ref, u_ref, v_ref, d_ref):
    b = pl.program_id(0)
    q = xp_ref[...]
    xt = xpt_ref[...]
    sq = jnp.sum(xt * xt, axis=0, keepdims=True)
    sqq = jnp.sum(q * q, axis=1, keepdims=True)
    d = sqq + sq - 2.0 * jnp.dot(q, xt, preferred_element_type=jnp.float32,
                                 precision=_HI)
    inf = jnp.float32(jnp.inf)
    coli = lax.broadcasted_iota(jnp.int32, (NB, NP), 1)
    rowg = lax.broadcasted_iota(jnp.int32, (NB, NP), 0) + (b * NB)
    d_ref[...] = jnp.where(coli == rowg, inf, d)
    lane = lax.broadcasted_iota(jnp.int32, (NB, 128), 1)

    NT = NP // 128
    lanef = lane.astype(jnp.float32)
    nbr = jnp.zeros((NB, 128), jnp.float32)
    idx_i = None
    for k in range(K):
        V = jnp.full((NB, 128), inf, jnp.float32)
        T = jnp.zeros((NB, 128), jnp.float32)
        for t in range(NT):
            x = d_ref[:, t * 128:(t + 1) * 128]
            if k > 0:
                x = jnp.where((lane + t * 128) == idx_i, inf, x)
                if k < K - 1:
                    d_ref[:, t * 128:(t + 1) * 128] = x
            mlt = x < V
            T = jnp.where(mlt, jnp.float32(t), T)
            V = jnp.where(mlt, x, V)
        mn = jnp.min(V, axis=1, keepdims=True)
        gidx = T * 128.0 + lanef
        cand = jnp.where(V == mn, gidx, jnp.float32(1e9))
        idxf = jnp.min(cand, axis=1, keepdims=True)
        nbr = jnp.where(lane == k, idxf, nbr)
        idx_i = idxf.astype(jnp.int32)
    nbr_ref[...] = nbr.astype(jnp.int32)
    u_ref[...] = jnp.dot(q, wu_ref[...], preferred_element_type=jnp.float32,
                         precision=_HI)
    v_ref[...] = jnp.dot(q, wv_ref[...], preferred_element_type=jnp.float32,
                         precision=_HI)


def _l1_body(e1_ref, u_ref, v_ref, xp_ref, w2_ref, w3a_ref, w3b_ref,
             b1_ref, b2_ref, g_ref, w_ref):
    v = v_ref[...]
    b1 = b1_ref[...]
    w2 = w2_ref[...]
    acc = jnp.dot(jnp.maximum(u_ref[...] - v + b1, 0.0), w2,
                  preferred_element_type=jnp.float32, precision=_HI)
    for k in range(K):
        uk = e1_ref[:, k * 32:(k + 1) * 32]
        hk = jnp.dot(jnp.maximum(uk - v + b1, 0.0), w2,
                     preferred_element_type=jnp.float32, precision=_HI)
        acc = jnp.maximum(acc, hk)
    h1 = jnp.maximum(acc + b2_ref[...], 0.0)
    wv = jnp.dot(xp_ref[...], w3b_ref[...], preferred_element_type=jnp.float32,
                 precision=_HI)
    g_ref[...] = jnp.dot(h1, w3a_ref[...], preferred_element_type=jnp.float32,
                         precision=_HI) + wv
    w_ref[...] = wv


def _l2_body(e2_ref, g_ref, w_ref, w4_ref, wc_ref, b3_ref, b4_ref, bc_ref,
             out_ref):
    w = w_ref[...]
    b3 = b3_ref[...]
    w4 = w4_ref[...]
    acc = jnp.dot(jnp.maximum(g_ref[...] - w + b3, 0.0), w4,
                  preferred_element_type=jnp.float32, precision=_HI)
    for k in range(K):
        gk = e2_ref[:, k * 64:(k + 1) * 64]
        hk = jnp.dot(jnp.maximum(gk - w + b3, 0.0), w4,
                     preferred_element_type=jnp.float32, precision=_HI)
        acc = jnp.maximum(acc, hk)
    h2 = jnp.maximum(acc + b4_ref[...], 0.0)
    logits = jnp.dot(h2, wc_ref[...], preferred_element_type=jnp.float32,
                     precision=_HI) + bc_ref[...]
    colk = lax.broadcasted_iota(jnp.int32, (NB, 128), 1)
    valid = colk < NUM_CLASSES
    lm = jnp.where(valid, logits, jnp.float32(-1e30))
    m = jnp.max(lm, axis=1, keepdims=True)
    e = jnp.where(valid, jnp.exp(logits - m), 0.0)
    s = jnp.sum(e, axis=1, keepdims=True)
    out_ref[...] = logits - m - jnp.log(s)


def _sc_gather(table, idx3d, d):
    mesh = plsc.VectorSubcoreMesh(core_axis_name="c", subcore_axis_name="s")
    NBUF = 4

    @functools.partial(
        pl.kernel, mesh=mesh,
        compiler_params=pltpu.CompilerParams(use_tc_tiling_on_sc=False),
        out_type=jax.ShapeDtypeStruct((NE_PAD, d), jnp.float32),
        scratch_types=[
            pltpu.VMEM((CHUNKS_PW, CHUNK), jnp.int32),
            pltpu.VMEM((NBUF, CHUNK, d), jnp.float32),
            pltpu.SemaphoreType.DMA((NBUF,)),
            pltpu.SemaphoreType.DMA((NBUF,)),
        ],
    )
    def k(table_hbm, idx_hbm, out_hbm, idx_v, rows, gsem, wsem):
        wid = lax.axis_index("s") * 2 + lax.axis_index("c")
        base = wid * CHUNKS_PW
        pltpu.sync_copy(idx_hbm.at[wid], idx_v)
        gh = [None] * CHUNKS_PW
        wh = [None] * CHUNKS_PW
        for j in range(CHUNKS_PW):
            b = j % NBUF
            if j >= NBUF:
                wh[j - NBUF].wait()
            gh[j] = pltpu.async_copy(table_hbm.at[idx_v.at[j]], rows.at[b],
                                     gsem.at[b])
            i = j - (NBUF - 1)
            if i >= 0:
                gh[i].wait()
                wh[i] = pltpu.async_copy(
                    rows.at[i % NBUF],
                    out_hbm.at[pl.ds((base + i) * CHUNK, CHUNK)],
                    wsem.at[i % NBUF])
        for i in range(CHUNKS_PW - (NBUF - 1), CHUNKS_PW):
            gh[i].wait()
            wh[i] = pltpu.async_copy(
                rows.at[i % NBUF],
                out_hbm.at[pl.ds((base + i) * CHUNK, CHUNK)],
                wsem.at[i % NBUF])
        for i in range(CHUNKS_PW - NBUF, CHUNKS_PW):
            wh[i].wait()

    return k(table, idx3d)


def _blk(shape, imap):
    return pl.BlockSpec(shape, imap)


def kernel(x, batch, W1, b1, W2, b2, W3, b3, W4, b4, Wc, bc):
    f32 = jnp.float32
    xp = jnp.zeros((NP, 8), f32).at[:N, :3].set(x)
    xpt = jnp.concatenate(
        [jnp.concatenate([x.T, jnp.full((3, NP - N), 1e18, f32)], axis=1),
         jnp.zeros((5, NP), f32)], axis=0)
    Wu = jnp.zeros((8, 32), f32).at[:3].set(W1[:3] + W1[3:6])
    Wv = jnp.zeros((8, 32), f32).at[:3].set(W1[3:6])
    W3a = W3[:64]
    W3bp = jnp.zeros((8, 64), f32).at[:3].set(W3[64:67])
    Wcp = jnp.zeros((128, 128), f32).at[:, :NUM_CLASSES].set(Wc)
    bcp = jnp.zeros((128,), f32).at[:NUM_CLASSES].set(bc)

    nbr_f, u, v = pl.pallas_call(
        _knn_body,
        grid=(NBLK,),
        in_specs=[
            _blk((NB, 8), lambda b: (b, 0)),
            _blk((8, NP), lambda b: (0, 0)),
            _blk((8, 32), lambda b: (0, 0)),
            _blk((8, 32), lambda b: (0, 0)),
        ],
        out_specs=[
            _blk((NB, 128), lambda b: (b, 0)),
            _blk((NB, 32), lambda b: (b, 0)),
            _blk((NB, 32), lambda b: (b, 0)),
        ],
        out_shape=[
            jax.ShapeDtypeStruct((NP, 128), jnp.int32),
            jax.ShapeDtypeStruct((NP, 32), f32),
            jax.ShapeDtypeStruct((NP, 32), f32),
        ],
        scratch_shapes=[pltpu.VMEM((NB, NP), jnp.float32)],
    )(xp, xpt, Wu, Wv)

    idx = nbr_f[:N, :K].reshape(-1)
    idx = jnp.concatenate(
        [idx, jnp.zeros((NE_PAD - N * K,), jnp.int32)]).reshape(
            NW, CHUNKS_PW, CHUNK)

    e1 = _sc_gather(u, idx, 32).reshape(-1, K * 32)
    g, w = pl.pallas_call(
        _l1_body,
        grid=(NBLK,),
        in_specs=[
            _blk((NB, K * 32), lambda b: (b, 0)),
            _blk((NB, 32), lambda b: (b, 0)),
            _blk((NB, 32), lambda b: (b, 0)),
            _blk((NB, 8), lambda b: (b, 0)),
            _blk((32, 64), lambda b: (0, 0)),
            _blk((64, 64), lambda b: (0, 0)),
            _blk((8, 64), lambda b: (0, 0)),
            _blk((1, 32), lambda b: (0, 0)),
            _blk((1, 64), lambda b: (0, 0)),
        ],
        out_specs=[
            _blk((NB, 64), lambda b: (b, 0)),
            _blk((NB, 64), lambda b: (b, 0)),
        ],
        out_shape=[
            jax.ShapeDtypeStruct((NP, 64), f32),
            jax.ShapeDtypeStruct((NP, 64), f32),
        ],
    )(e1, u, v, xp, W2, W3a, W3bp, b1.reshape(1, 32), b2.reshape(1, 64))

    e2 = _sc_gather(g, idx, 64).reshape(-1, K * 64)
    out = pl.pallas_call(
        _l2_body,
        grid=(NBLK,),
        in_specs=[
            _blk((NB, K * 64), lambda b: (b, 0)),
            _blk((NB, 64), lambda b: (b, 0)),
            _blk((NB, 64), lambda b: (b, 0)),
            _blk((64, 128), lambda b: (0, 0)),
            _blk((128, 128), lambda b: (0, 0)),
            _blk((1, 64), lambda b: (0, 0)),
            _blk((1, 128), lambda b: (0, 0)),
            _blk((1, 128), lambda b: (0, 0)),
        ],
        out_specs=_blk((NB, 128), lambda b: (b, 0)),
        out_shape=jax.ShapeDtypeStruct((NP, 128), f32),
    )(e2, g, w, W4, Wcp, b3.reshape(1, 64), b4.reshape(1, 128),
      bcp.reshape(1, 128))

    return out[:N, :NUM_CLASSES]


# speedup vs baseline: 1.2421x; 1.0255x over previous
import functools

import jax
import jax.numpy as jnp
from jax import lax
from jax.experimental import pallas as pl
from jax.experimental.pallas import tpu as pltpu
from jax.experimental.pallas import tpu_sc as plsc

N = 10000
K = 8
NUM_CLASSES = 40

NB = 128
NBLK = 79
NP = NB * NBLK
NW = 32
CHUNK = 128
CHUNKS_PW = 20
NE_PAD = NW * CHUNKS_PW * CHUNK

_HI = jax.lax.Precision.HIGHEST


def _knn_body(xp_ref, xpt_ref, wu_ref, wv_ref, nbr_ref, u_ref, v_ref, d_ref):
    b = pl.program_id(0)
    q = xp_ref[...]
    xt = xpt_ref[...]
    sq = jnp.sum(xt * xt, axis=0, keepdims=True)
    sqq = jnp.sum(q * q, axis=1, keepdims=True)
    d = sqq + sq - 2.0 * jnp.dot(q, xt, preferred_element_type=jnp.float32,
                                 precision=_HI)
    inf = jnp.float32(jnp.inf)
    coli = lax.broadcasted_iota(jnp.int32, (NB, NP), 1)
    rowg = lax.broadcasted_iota(jnp.int32, (NB, NP), 0) + (b * NB)
    d_ref[...] = jnp.where(coli == rowg, inf, d)
    lane = lax.broadcasted_iota(jnp.int32, (NB, 128), 1)

    NT = NP // 128
    lanef = lane.astype(jnp.float32)
    nbr = jnp.zeros((NB, 128), jnp.float32)
    idx_i = None
    for k in range(K):
        V = jnp.full((NB, 128), inf, jnp.float32)
        T = jnp.zeros((NB, 128), jnp.float32)
        for t in range(NT):
            x = d_ref[:, t * 128:(t + 1) * 128]
            if k > 0:
                x = jnp.where((lane + t * 128) == idx_i, inf, x)
                if k < K - 1:
                    d_ref[:, t * 128:(t + 1) * 128] = x
            mlt = x < V
            T = jnp.where(mlt, jnp.float32(t), T)
            V = jnp.where(mlt, x, V)
        mn = jnp.min(V, axis=1, keepdims=True)
        gidx = T * 128.0 + lanef
        cand = jnp.where(V == mn, gidx, jnp.float32(1e9))
        idxf = jnp.min(cand, axis=1, keepdims=True)
        nbr = jnp.where(lane == k, idxf, nbr)
        idx_i = idxf.astype(jnp.int32)
    nbr_ref[...] = nbr.astype(jnp.int32)
    u_ref[...] = jnp.dot(q, wu_ref[...], preferred_element_type=jnp.float32,
                         precision=_HI)
    v_ref[...] = jnp.dot(q, wv_ref[...], preferred_element_type=jnp.float32,
                         precision=_HI)


def _l1_body(e1_ref, u_ref, v_ref, xp_ref, w2_ref, w3a_ref, w3b_ref,
             b1_ref, b2_ref, g_ref, w_ref):
    v = v_ref[...]
    b1 = b1_ref[...]
    w2 = w2_ref[...]
    acc = jnp.dot(jnp.maximum(u_ref[...] - v + b1, 0.0), w2,
                  preferred_element_type=jnp.float32, precision=_HI)
    for k in range(K):
        uk = e1_ref[:, k * 32:(k + 1) * 32]
        hk = jnp.dot(jnp.maximum(uk - v + b1, 0.0), w2,
                     preferred_element_type=jnp.float32, precision=_HI)
        acc = jnp.maximum(acc, hk)
    h1 = jnp.maximum(acc + b2_ref[...], 0.0)
    wv = jnp.dot(xp_ref[...], w3b_ref[...], preferred_element_type=jnp.float32,
                 precision=_HI)
    g_ref[...] = jnp.dot(h1, w3a_ref[...], preferred_element_type=jnp.float32,
                         precision=_HI) + wv
    w_ref[...] = wv


def _l2_body(e2_ref, g_ref, w_ref, w4_ref, wc_ref, b3_ref, b4_ref, bc_ref,
             out_ref):
    rows = w_ref.shape[0]
    w = w_ref[...]
    b3 = b3_ref[...]
    w4 = w4_ref[...]
    acc = jnp.dot(jnp.maximum(g_ref[...] - w + b3, 0.0), w4,
                  preferred_element_type=jnp.float32, precision=_HI)
    for k in range(K):
        gk = e2_ref[:, k * 64:(k + 1) * 64]
        hk = jnp.dot(jnp.maximum(gk - w + b3, 0.0), w4,
                     preferred_element_type=jnp.float32, precision=_HI)
        acc = jnp.maximum(acc, hk)
    h2 = jnp.maximum(acc + b4_ref[...], 0.0)
    logits = jnp.dot(h2, wc_ref[...], preferred_element_type=jnp.float32,
                     precision=_HI) + bc_ref[...]
    colk = lax.broadcasted_iota(jnp.int32, (rows, 128), 1)
    valid = colk < NUM_CLASSES
    lm = jnp.where(valid, logits, jnp.float32(-1e30))
    m = jnp.max(lm, axis=1, keepdims=True)
    e = jnp.where(valid, jnp.exp(logits - m), 0.0)
    s = jnp.sum(e, axis=1, keepdims=True)
    out_ref[...] = logits - m - jnp.log(s)


def _sc_gather(table, idx3d, d):
    mesh = plsc.VectorSubcoreMesh(core_axis_name="c", subcore_axis_name="s")
    NBUF = 4

    @functools.partial(
        pl.kernel, mesh=mesh,
        compiler_params=pltpu.CompilerParams(use_tc_tiling_on_sc=False),
        out_type=jax.ShapeDtypeStruct((NE_PAD, d), jnp.float32),
        scratch_types=[
            pltpu.VMEM((CHUNKS_PW, CHUNK), jnp.int32),
            pltpu.VMEM((NBUF, CHUNK, d), jnp.float32),
            pltpu.SemaphoreType.DMA((NBUF,)),
            pltpu.SemaphoreType.DMA((NBUF,)),
        ],
    )
    def k(table_hbm, idx_hbm, out_hbm, idx_v, rows, gsem, wsem):
        wid = lax.axis_index("s") * 2 + lax.axis_index("c")
        base = wid * CHUNKS_PW
        pltpu.sync_copy(idx_hbm.at[wid], idx_v)
        gh = [None] * CHUNKS_PW
        wh = [None] * CHUNKS_PW
        for j in range(CHUNKS_PW):
            b = j % NBUF
            if j >= NBUF:
                wh[j - NBUF].wait()
            gh[j] = pltpu.async_copy(table_hbm.at[idx_v.at[j]], rows.at[b],
                                     gsem.at[b])
            i = j - (NBUF - 1)
            if i >= 0:
                gh[i].wait()
                wh[i] = pltpu.async_copy(
                    rows.at[i % NBUF],
                    out_hbm.at[pl.ds((base + i) * CHUNK, CHUNK)],
                    wsem.at[i % NBUF])
        for i in range(CHUNKS_PW - (NBUF - 1), CHUNKS_PW):
            gh[i].wait()
            wh[i] = pltpu.async_copy(
                rows.at[i % NBUF],
                out_hbm.at[pl.ds((base + i) * CHUNK, CHUNK)],
                wsem.at[i % NBUF])
        for i in range(CHUNKS_PW - NBUF, CHUNKS_PW):
            wh[i].wait()

    return k(table, idx3d)


def _blk(shape, imap):
    return pl.BlockSpec(shape, imap)


def kernel(x, batch, W1, b1, W2, b2, W3, b3, W4, b4, Wc, bc):
    f32 = jnp.float32
    xp = jnp.zeros((NP, 8), f32).at[:N, :3].set(x)
    xpt = jnp.concatenate(
        [jnp.concatenate([x.T, jnp.full((3, NP - N), 1e18, f32)], axis=1),
         jnp.zeros((5, NP), f32)], axis=0)
    Wu = jnp.zeros((8, 32), f32).at[:3].set(W1[:3] + W1[3:6])
    Wv = jnp.zeros((8, 32), f32).at[:3].set(W1[3:6])
    W3a = W3[:64]
    W3bp = jnp.zeros((8, 64), f32).at[:3].set(W3[64:67])
    Wcp = jnp.zeros((128, 128), f32).at[:, :NUM_CLASSES].set(Wc)
    bcp = jnp.zeros((128,), f32).at[:NUM_CLASSES].set(bc)

    nbr_f, u, v = pl.pallas_call(
        _knn_body,
        grid=(NBLK,),
        in_specs=[
            _blk((NB, 8), lambda b: (b, 0)),
            _blk((8, NP), lambda b: (0, 0)),
            _blk((8, 32), lambda b: (0, 0)),
            _blk((8, 32), lambda b: (0, 0)),
        ],
        out_specs=[
            _blk((NB, 128), lambda b: (b, 0)),
            _blk((NB, 32), lambda b: (b, 0)),
            _blk((NB, 32), lambda b: (b, 0)),
        ],
        out_shape=[
            jax.ShapeDtypeStruct((NP, 128), jnp.int32),
            jax.ShapeDtypeStruct((NP, 32), f32),
            jax.ShapeDtypeStruct((NP, 32), f32),
        ],
        scratch_shapes=[pltpu.VMEM((NB, NP), jnp.float32)],
    )(xp, xpt, Wu, Wv)

    idx = nbr_f[:N, :K].reshape(-1)
    idx = jnp.concatenate(
        [idx, jnp.zeros((NE_PAD - N * K,), jnp.int32)]).reshape(
            NW, CHUNKS_PW, CHUNK)

    NP2 = NE_PAD // K
    NB2 = 256
    NBLK2 = NP2 // NB2
    u2 = jnp.zeros((NP2, 32), f32).at[:NP].set(u)
    v2 = jnp.zeros((NP2, 32), f32).at[:NP].set(v)
    xp2 = jnp.zeros((NP2, 8), f32).at[:NP].set(xp)

    e1 = _sc_gather(u, idx, 32).reshape(-1, K * 32)
    g, w = pl.pallas_call(
        _l1_body,
        grid=(NBLK2,),
        in_specs=[
            _blk((NB2, K * 32), lambda b: (b, 0)),
            _blk((NB2, 32), lambda b: (b, 0)),
            _blk((NB2, 32), lambda b: (b, 0)),
            _blk((NB2, 8), lambda b: (b, 0)),
            _blk((32, 64), lambda b: (0, 0)),
            _blk((64, 64), lambda b: (0, 0)),
            _blk((8, 64), lambda b: (0, 0)),
            _blk((1, 32), lambda b: (0, 0)),
            _blk((1, 64), lambda b: (0, 0)),
        ],
        out_specs=[
            _blk((NB2, 64), lambda b: (b, 0)),
            _blk((NB2, 64), lambda b: (b, 0)),
        ],
        out_shape=[
            jax.ShapeDtypeStruct((NP2, 64), f32),
            jax.ShapeDtypeStruct((NP2, 64), f32),
        ],
    )(e1, u2, v2, xp2, W2, W3a, W3bp, b1.reshape(1, 32), b2.reshape(1, 64))

    e2 = _sc_gather(g, idx, 64).reshape(-1, K * 64)
    out = pl.pallas_call(
        _l2_body,
        grid=(NBLK2,),
        in_specs=[
            _blk((NB2, K * 64), lambda b: (b, 0)),
            _blk((NB2, 64), lambda b: (b, 0)),
            _blk((NB2, 64), lambda b: (b, 0)),
            _blk((64, 128), lambda b: (0, 0)),
            _blk((128, 128), lambda b: (0, 0)),
            _blk((1, 64), lambda b: (0, 0)),
            _blk((1, 128), lambda b: (0, 0)),
            _blk((1, 128), lambda b: (0, 0)),
        ],
        out_specs=_blk((NB2, 128), lambda b: (b, 0)),
        out_shape=jax.ShapeDtypeStruct((NP2, 128), f32),
    )(e2, g, w, W4, Wcp, b3.reshape(1, 64), b4.reshape(1, 128),
      bcp.reshape(1, 128))

    return out[:N, :NUM_CLASSES]


# speedup vs baseline: 1.2607x; 1.0150x over previous
import functools

import jax
import jax.numpy as jnp
from jax import lax
from jax.experimental import pallas as pl
from jax.experimental.pallas import tpu as pltpu
from jax.experimental.pallas import tpu_sc as plsc

N = 10000
K = 8
NUM_CLASSES = 40

NB = 128
NBLK = 79
NP = NB * NBLK
NW = 32
CHUNK = 128
CHUNKS_PW = 20
NE_PAD = NW * CHUNKS_PW * CHUNK

_HI = jax.lax.Precision.HIGHEST


def _knn_body(xp_ref, xpt_ref, wu_ref, wv_ref, nbr_ref, u_ref, v_ref, d_ref):
    b = pl.program_id(0)
    q = xp_ref[...]
    xt = xpt_ref[...]
    sq = jnp.sum(xt * xt, axis=0, keepdims=True)
    sqq = jnp.sum(q * q, axis=1, keepdims=True)
    d = sqq + sq - 2.0 * jnp.dot(q, xt, preferred_element_type=jnp.float32,
                                 precision=_HI)
    inf = jnp.float32(jnp.inf)
    coli = lax.broadcasted_iota(jnp.int32, (NB, NP), 1)
    rowg = lax.broadcasted_iota(jnp.int32, (NB, NP), 0) + (b * NB)
    d_ref[...] = jnp.where(coli == rowg, inf, d)
    lane = lax.broadcasted_iota(jnp.int32, (NB, 128), 1)

    NT = NP // 128
    lanef = lane.astype(jnp.float32)
    nbr = jnp.zeros((NB, 128), jnp.float32)
    idx_i = None
    for k in range(K):
        V = jnp.full((NB, 128), inf, jnp.float32)
        T = jnp.zeros((NB, 128), jnp.float32)
        for t in range(NT):
            x = d_ref[:, t * 128:(t + 1) * 128]
            if k > 0:
                x = jnp.where((lane + t * 128) == idx_i, inf, x)
                if k < K - 1:
                    d_ref[:, t * 128:(t + 1) * 128] = x
            mlt = x < V
            T = jnp.where(mlt, jnp.float32(t), T)
            V = jnp.where(mlt, x, V)
        mn = jnp.min(V, axis=1, keepdims=True)
        gidx = T * 128.0 + lanef
        cand = jnp.where(V == mn, gidx, jnp.float32(1e9))
        idxf = jnp.min(cand, axis=1, keepdims=True)
        nbr = jnp.where(lane == k, idxf, nbr)
        idx_i = idxf.astype(jnp.int32)
    nbr_ref[...] = nbr.astype(jnp.int32)
    u_ref[...] = jnp.dot(q, wu_ref[...], preferred_element_type=jnp.float32,
                         precision=_HI)
    v_ref[...] = jnp.dot(q, wv_ref[...], preferred_element_type=jnp.float32,
                         precision=_HI)


def _l1_body(e1_ref, u_ref, v_ref, xp_ref, w2_ref, w3a_ref, w3b_ref,
             b1_ref, b2_ref, g_ref, w_ref):
    v = v_ref[...]
    b1 = b1_ref[...]
    w2 = w2_ref[...]
    acc = jnp.dot(jnp.maximum(u_ref[...] - v + b1, 0.0), w2,
                  preferred_element_type=jnp.float32, precision=_HI)
    for k in range(K):
        uk = e1_ref[:, k * 32:(k + 1) * 32]
        hk = jnp.dot(jnp.maximum(uk - v + b1, 0.0), w2,
                     preferred_element_type=jnp.float32, precision=_HI)
        acc = jnp.maximum(acc, hk)
    h1 = jnp.maximum(acc + b2_ref[...], 0.0)
    wv = jnp.dot(xp_ref[...], w3b_ref[...], preferred_element_type=jnp.float32,
                 precision=_HI)
    g_ref[...] = jnp.dot(h1, w3a_ref[...], preferred_element_type=jnp.float32,
                         precision=_HI) + wv
    w_ref[...] = wv


def _l2_body(e2_ref, g_ref, w_ref, w4_ref, wc_ref, b3_ref, b4_ref, bc_ref,
             out_ref):
    rows = w_ref.shape[0]
    w = w_ref[...]
    b3 = b3_ref[...]
    w4 = w4_ref[...]
    acc = jnp.dot(jnp.maximum(g_ref[...] - w + b3, 0.0), w4,
                  preferred_element_type=jnp.float32, precision=_HI)
    for k in range(K):
        gk = e2_ref[:, k * 64:(k + 1) * 64]
        hk = jnp.dot(jnp.maximum(gk - w + b3, 0.0), w4,
                     preferred_element_type=jnp.float32, precision=_HI)
        acc = jnp.maximum(acc, hk)
    h2 = jnp.maximum(acc + b4_ref[...], 0.0)
    logits = jnp.dot(h2, wc_ref[...], preferred_element_type=jnp.float32,
                     precision=_HI) + bc_ref[...]
    colk = lax.broadcasted_iota(jnp.int32, (rows, 128), 1)
    valid = colk < NUM_CLASSES
    lm = jnp.where(valid, logits, jnp.float32(-1e30))
    m = jnp.max(lm, axis=1, keepdims=True)
    e = jnp.where(valid, jnp.exp(logits - m), 0.0)
    s = jnp.sum(e, axis=1, keepdims=True)
    out_ref[...] = logits - m - jnp.log(s)


def _sc_gather(table, idx3d, d):
    mesh = plsc.VectorSubcoreMesh(core_axis_name="c", subcore_axis_name="s")
    NBUF = 4

    @functools.partial(
        pl.kernel, mesh=mesh,
        compiler_params=pltpu.CompilerParams(use_tc_tiling_on_sc=False),
        out_type=jax.ShapeDtypeStruct((NE_PAD, d), jnp.float32),
        scratch_types=[
            pltpu.VMEM((CHUNKS_PW, CHUNK), jnp.int32),
            pltpu.VMEM((NBUF, CHUNK, d), jnp.float32),
            pltpu.SemaphoreType.DMA((NBUF,)),
            pltpu.SemaphoreType.DMA((NBUF,)),
        ],
    )
    def k(table_hbm, idx_hbm, out_hbm, idx_v, rows, gsem, wsem):
        wid = lax.axis_index("s") * 2 + lax.axis_index("c")
        base = wid * CHUNKS_PW
        pltpu.sync_copy(idx_hbm.at[wid], idx_v)
        gh = [None] * CHUNKS_PW
        wh = [None] * CHUNKS_PW
        for j in range(CHUNKS_PW):
            b = j % NBUF
            if j >= NBUF:
                wh[j - NBUF].wait()
            gh[j] = pltpu.async_copy(table_hbm.at[idx_v.at[j]], rows.at[b],
                                     gsem.at[b])
            i = j - (NBUF - 1)
            if i >= 0:
                gh[i].wait()
                wh[i] = pltpu.async_copy(
                    rows.at[i % NBUF],
                    out_hbm.at[pl.ds((base + i) * CHUNK, CHUNK)],
                    wsem.at[i % NBUF])
        for i in range(CHUNKS_PW - (NBUF - 1), CHUNKS_PW):
            gh[i].wait()
            wh[i] = pltpu.async_copy(
                rows.at[i % NBUF],
                out_hbm.at[pl.ds((base + i) * CHUNK, CHUNK)],
                wsem.at[i % NBUF])
        for i in range(CHUNKS_PW - NBUF, CHUNKS_PW):
            wh[i].wait()

    return k(table, idx3d)


def _blk(shape, imap):
    return pl.BlockSpec(shape, imap)


def kernel(x, batch, W1, b1, W2, b2, W3, b3, W4, b4, Wc, bc):
    f32 = jnp.float32
    xp = jnp.zeros((NP, 8), f32).at[:N, :3].set(x)
    xpt = jnp.concatenate(
        [jnp.concatenate([x.T, jnp.full((3, NP - N), 1e18, f32)], axis=1),
         jnp.zeros((5, NP), f32)], axis=0)
    Wu = jnp.zeros((8, 32), f32).at[:3].set(W1[:3] + W1[3:6])
    Wv = jnp.zeros((8, 32), f32).at[:3].set(W1[3:6])
    W3a = W3[:64]
    W3bp = jnp.zeros((8, 64), f32).at[:3].set(W3[64:67])
    Wcp = jnp.zeros((128, 128), f32).at[:, :NUM_CLASSES].set(Wc)
    bcp = jnp.zeros((128,), f32).at[:NUM_CLASSES].set(bc)

    nbr_f, u, v = pl.pallas_call(
        _knn_body,
        grid=(NBLK,),
        in_specs=[
            _blk((NB, 8), lambda b: (b, 0)),
            _blk((8, NP), lambda b: (0, 0)),
            _blk((8, 32), lambda b: (0, 0)),
            _blk((8, 32), lambda b: (0, 0)),
        ],
        out_specs=[
            _blk((NB, 128), lambda b: (b, 0)),
            _blk((NB, 32), lambda b: (b, 0)),
            _blk((NB, 32), lambda b: (b, 0)),
        ],
        out_shape=[
            jax.ShapeDtypeStruct((NP, 128), jnp.int32),
            jax.ShapeDtypeStruct((NP, 32), f32),
            jax.ShapeDtypeStruct((NP, 32), f32),
        ],
        scratch_shapes=[pltpu.VMEM((NB, NP), jnp.float32)],
    )(xp, xpt, Wu, Wv)

    idx = nbr_f[:N, :K].reshape(-1)
    idx = jnp.concatenate(
        [idx, jnp.zeros((NE_PAD - N * K,), jnp.int32)]).reshape(
            NW, CHUNKS_PW, CHUNK)

    NP2 = NE_PAD // K
    NB2 = 512
    NBLK2 = NP2 // NB2
    u2 = jnp.zeros((NP2, 32), f32).at[:NP].set(u)
    v2 = jnp.zeros((NP2, 32), f32).at[:NP].set(v)
    xp2 = jnp.zeros((NP2, 8), f32).at[:NP].set(xp)

    e1 = _sc_gather(u, idx, 32).reshape(-1, K * 32)
    g, w = pl.pallas_call(
        _l1_body,
        grid=(NBLK2,),
        in_specs=[
            _blk((NB2, K * 32), lambda b: (b, 0)),
            _blk((NB2, 32), lambda b: (b, 0)),
            _blk((NB2, 32), lambda b: (b, 0)),
            _blk((NB2, 8), lambda b: (b, 0)),
            _blk((32, 64), lambda b: (0, 0)),
            _blk((64, 64), lambda b: (0, 0)),
            _blk((8, 64), lambda b: (0, 0)),
            _blk((1, 32), lambda b: (0, 0)),
            _blk((1, 64), lambda b: (0, 0)),
        ],
        out_specs=[
            _blk((NB2, 64), lambda b: (b, 0)),
            _blk((NB2, 64), lambda b: (b, 0)),
        ],
        out_shape=[
            jax.ShapeDtypeStruct((NP2, 64), f32),
            jax.ShapeDtypeStruct((NP2, 64), f32),
        ],
    )(e1, u2, v2, xp2, W2, W3a, W3bp, b1.reshape(1, 32), b2.reshape(1, 64))

    e2 = _sc_gather(g, idx, 64).reshape(-1, K * 64)
    out = pl.pallas_call(
        _l2_body,
        grid=(NBLK2,),
        in_specs=[
            _blk((NB2, K * 64), lambda b: (b, 0)),
            _blk((NB2, 64), lambda b: (b, 0)),
            _blk((NB2, 64), lambda b: (b, 0)),
            _blk((64, 128), lambda b: (0, 0)),
            _blk((128, 128), lambda b: (0, 0)),
            _blk((1, 64), lambda b: (0, 0)),
            _blk((1, 128), lambda b: (0, 0)),
            _blk((1, 128), lambda b: (0, 0)),
        ],
        out_specs=_blk((NB2, 128), lambda b: (b, 0)),
        out_shape=jax.ShapeDtypeStruct((NP2, 128), f32),
    )(e2, g, w, W4, Wcp, b3.reshape(1, 64), b4.reshape(1, 128),
      bcp.reshape(1, 128))

    return out[:N, :NUM_CLASSES]


# speedup vs baseline: 1.2654x; 1.0038x over previous
import functools

import jax
import jax.numpy as jnp
from jax import lax
from jax.experimental import pallas as pl
from jax.experimental.pallas import tpu as pltpu
from jax.experimental.pallas import tpu_sc as plsc

N = 10000
K = 8
NUM_CLASSES = 40

NB = 128
NBLK = 79
NP = NB * NBLK
NW = 32
CHUNK = 128
CHUNKS_PW = 20
NE_PAD = NW * CHUNKS_PW * CHUNK

_HI = jax.lax.Precision.HIGHEST


def _knn_body(xp_ref, xpt_ref, wu_ref, wv_ref, nbr_ref, u_ref, v_ref, d_ref):
    b = pl.program_id(0)
    q = xp_ref[...]
    xt = xpt_ref[...]
    sq = jnp.sum(xt * xt, axis=0, keepdims=True)
    sqq = jnp.sum(q * q, axis=1, keepdims=True)
    d = sqq + sq - 2.0 * jnp.dot(q, xt, preferred_element_type=jnp.float32,
                                 precision=_HI)
    inf = jnp.float32(jnp.inf)
    coli = lax.broadcasted_iota(jnp.int32, (NB, NP), 1)
    rowg = lax.broadcasted_iota(jnp.int32, (NB, NP), 0) + (b * NB)
    d_ref[...] = jnp.where(coli == rowg, inf, d)
    lane = lax.broadcasted_iota(jnp.int32, (NB, 128), 1)

    NT = NP // 128
    lanef = lane.astype(jnp.float32)
    nbr = jnp.zeros((NB, 128), jnp.float32)
    idx_i = None
    for k in range(K):
        V = jnp.full((NB, 128), inf, jnp.float32)
        T = jnp.zeros((NB, 128), jnp.float32)
        for t in range(NT):
            x = d_ref[:, t * 128:(t + 1) * 128]
            if k > 0:
                x = jnp.where((lane + t * 128) == idx_i, inf, x)
                if k < K - 1:
                    d_ref[:, t * 128:(t + 1) * 128] = x
            mlt = x < V
            T = jnp.where(mlt, jnp.float32(t), T)
            V = jnp.where(mlt, x, V)
        mn = jnp.min(V, axis=1, keepdims=True)
        gidx = T * 128.0 + lanef
        cand = jnp.where(V == mn, gidx, jnp.float32(1e9))
        idxf = jnp.min(cand, axis=1, keepdims=True)
        nbr = jnp.where(lane == k, idxf, nbr)
        idx_i = idxf.astype(jnp.int32)
    nbr_ref[...] = nbr.astype(jnp.int32)
    u_ref[...] = jnp.dot(q, wu_ref[...], preferred_element_type=jnp.float32,
                         precision=_HI)
    v_ref[...] = jnp.dot(q, wv_ref[...], preferred_element_type=jnp.float32,
                         precision=_HI)


def _l1_body(e1_ref, u_ref, v_ref, xp_ref, w2_ref, w3a_ref, w3b_ref,
             b1_ref, b2_ref, g_ref, w_ref):
    v = v_ref[...]
    b1 = b1_ref[...]
    w2 = w2_ref[...]
    acc = jnp.dot(jnp.maximum(u_ref[...] - v + b1, 0.0), w2,
                  preferred_element_type=jnp.float32, precision=_HI)
    for k in range(K):
        uk = e1_ref[:, k * 32:(k + 1) * 32]
        hk = jnp.dot(jnp.maximum(uk - v + b1, 0.0), w2,
                     preferred_element_type=jnp.float32, precision=_HI)
        acc = jnp.maximum(acc, hk)
    h1 = jnp.maximum(acc + b2_ref[...], 0.0)
    wv = jnp.dot(xp_ref[...], w3b_ref[...], preferred_element_type=jnp.float32,
                 precision=_HI)
    g_ref[...] = jnp.dot(h1, w3a_ref[...], preferred_element_type=jnp.float32,
                         precision=_HI) + wv
    w_ref[...] = wv


def _l2_body(e2_ref, g_ref, w_ref, w4_ref, wc_ref, b3_ref, b4_ref, bc_ref,
             out_ref):
    rows = w_ref.shape[0]
    w = w_ref[...]
    b3 = b3_ref[...]
    w4 = w4_ref[...]
    acc = jnp.dot(jnp.maximum(g_ref[...] - w + b3, 0.0), w4,
                  preferred_element_type=jnp.float32, precision=_HI)
    for k in range(K):
        gk = e2_ref[:, k * 64:(k + 1) * 64]
        hk = jnp.dot(jnp.maximum(gk - w + b3, 0.0), w4,
                     preferred_element_type=jnp.float32, precision=_HI)
        acc = jnp.maximum(acc, hk)
    h2 = jnp.maximum(acc + b4_ref[...], 0.0)
    logits = jnp.dot(h2, wc_ref[...], preferred_element_type=jnp.float32,
                     precision=_HI) + bc_ref[...]
    colk = lax.broadcasted_iota(jnp.int32, (rows, 128), 1)
    valid = colk < NUM_CLASSES
    lm = jnp.where(valid, logits, jnp.float32(-1e30))
    m = jnp.max(lm, axis=1, keepdims=True)
    e = jnp.where(valid, jnp.exp(logits - m), 0.0)
    s = jnp.sum(e, axis=1, keepdims=True)
    out_ref[...] = logits - m - jnp.log(s)


def _sc_gather(table, idx3d, d):
    mesh = plsc.VectorSubcoreMesh(core_axis_name="c", subcore_axis_name="s")
    NBUF = 4

    @functools.partial(
        pl.kernel, mesh=mesh,
        compiler_params=pltpu.CompilerParams(use_tc_tiling_on_sc=False),
        out_type=jax.ShapeDtypeStruct((NE_PAD, d), jnp.float32),
        scratch_types=[
            pltpu.VMEM((CHUNKS_PW, CHUNK), jnp.int32),
            pltpu.VMEM((NBUF, CHUNK, d), jnp.float32),
            pltpu.SemaphoreType.DMA((NBUF,)),
            pltpu.SemaphoreType.DMA((NBUF,)),
        ],
    )
    def k(table_hbm, idx_hbm, out_hbm, idx_v, rows, gsem, wsem):
        wid = lax.axis_index("s") * 2 + lax.axis_index("c")
        base = wid * CHUNKS_PW
        pltpu.sync_copy(idx_hbm.at[wid], idx_v)
        gh = [None] * CHUNKS_PW
        wh = [None] * CHUNKS_PW
        for j in range(CHUNKS_PW):
            b = j % NBUF
            if j >= NBUF:
                wh[j - NBUF].wait()
            gh[j] = pltpu.async_copy(table_hbm.at[idx_v.at[j]], rows.at[b],
                                     gsem.at[b])
            i = j - (NBUF - 1)
            if i >= 0:
                gh[i].wait()
                wh[i] = pltpu.async_copy(
                    rows.at[i % NBUF],
                    out_hbm.at[pl.ds((base + i) * CHUNK, CHUNK)],
                    wsem.at[i % NBUF])
        for i in range(CHUNKS_PW - (NBUF - 1), CHUNKS_PW):
            gh[i].wait()
            wh[i] = pltpu.async_copy(
                rows.at[i % NBUF],
                out_hbm.at[pl.ds((base + i) * CHUNK, CHUNK)],
                wsem.at[i % NBUF])
        for i in range(CHUNKS_PW - NBUF, CHUNKS_PW):
            wh[i].wait()

    return k(table, idx3d)


def _blk(shape, imap):
    return pl.BlockSpec(shape, imap)


def kernel(x, batch, W1, b1, W2, b2, W3, b3, W4, b4, Wc, bc):
    f32 = jnp.float32
    xp = jnp.zeros((NP, 8), f32).at[:N, :3].set(x)
    xpt = jnp.concatenate(
        [jnp.concatenate([x.T, jnp.full((3, NP - N), 1e18, f32)], axis=1),
         jnp.zeros((5, NP), f32)], axis=0)
    Wu = jnp.zeros((8, 32), f32).at[:3].set(W1[:3] + W1[3:6])
    Wv = jnp.zeros((8, 32), f32).at[:3].set(W1[3:6])
    W3a = W3[:64]
    W3bp = jnp.zeros((8, 64), f32).at[:3].set(W3[64:67])
    Wcp = jnp.zeros((128, 128), f32).at[:, :NUM_CLASSES].set(Wc)
    bcp = jnp.zeros((128,), f32).at[:NUM_CLASSES].set(bc)

    nbr_f, u, v = pl.pallas_call(
        _knn_body,
        grid=(NBLK,),
        in_specs=[
            _blk((NB, 8), lambda b: (b, 0)),
            _blk((8, NP), lambda b: (0, 0)),
            _blk((8, 32), lambda b: (0, 0)),
            _blk((8, 32), lambda b: (0, 0)),
        ],
        out_specs=[
            _blk((NB, 128), lambda b: (b, 0)),
            _blk((NB, 32), lambda b: (b, 0)),
            _blk((NB, 32), lambda b: (b, 0)),
        ],
        out_shape=[
            jax.ShapeDtypeStruct((NP, 128), jnp.int32),
            jax.ShapeDtypeStruct((NP, 32), f32),
            jax.ShapeDtypeStruct((NP, 32), f32),
        ],
        scratch_shapes=[pltpu.VMEM((NB, NP), jnp.float32)],
    )(xp, xpt, Wu, Wv)

    idx = nbr_f[:N, :K].reshape(-1)
    idx = jnp.concatenate(
        [idx, jnp.zeros((NE_PAD - N * K,), jnp.int32)]).reshape(
            NW, CHUNKS_PW, CHUNK)

    NP2 = NE_PAD // K
    NB2 = 1024
    NBLK2 = NP2 // NB2
    u2 = jnp.zeros((NP2, 32), f32).at[:NP].set(u)
    v2 = jnp.zeros((NP2, 32), f32).at[:NP].set(v)
    xp2 = jnp.zeros((NP2, 8), f32).at[:NP].set(xp)

    e1 = _sc_gather(u, idx, 32).reshape(-1, K * 32)
    g, w = pl.pallas_call(
        _l1_body,
        grid=(NBLK2,),
        in_specs=[
            _blk((NB2, K * 32), lambda b: (b, 0)),
            _blk((NB2, 32), lambda b: (b, 0)),
            _blk((NB2, 32), lambda b: (b, 0)),
            _blk((NB2, 8), lambda b: (b, 0)),
            _blk((32, 64), lambda b: (0, 0)),
            _blk((64, 64), lambda b: (0, 0)),
            _blk((8, 64), lambda b: (0, 0)),
            _blk((1, 32), lambda b: (0, 0)),
            _blk((1, 64), lambda b: (0, 0)),
        ],
        out_specs=[
            _blk((NB2, 64), lambda b: (b, 0)),
            _blk((NB2, 64), lambda b: (b, 0)),
        ],
        out_shape=[
            jax.ShapeDtypeStruct((NP2, 64), f32),
            jax.ShapeDtypeStruct((NP2, 64), f32),
        ],
    )(e1, u2, v2, xp2, W2, W3a, W3bp, b1.reshape(1, 32), b2.reshape(1, 64))

    e2 = _sc_gather(g, idx, 64).reshape(-1, K * 64)
    out = pl.pallas_call(
        _l2_body,
        grid=(NBLK2,),
        in_specs=[
            _blk((NB2, K * 64), lambda b: (b, 0)),
            _blk((NB2, 64), lambda b: (b, 0)),
            _blk((NB2, 64), lambda b: (b, 0)),
            _blk((64, 128), lambda b: (0, 0)),
            _blk((128, 128), lambda b: (0, 0)),
            _blk((1, 64), lambda b: (0, 0)),
            _blk((1, 128), lambda b: (0, 0)),
            _blk((1, 128), lambda b: (0, 0)),
        ],
        out_specs=_blk((NB2, 128), lambda b: (b, 0)),
        out_shape=jax.ShapeDtypeStruct((NP2, 128), f32),
    )(e2, g, w, W4, Wcp, b3.reshape(1, 64), b4.reshape(1, 128),
      bcp.reshape(1, 128))

    return out[:N, :NUM_CLASSES]
